# baseline (speedup 1.0000x reference)
import functools

import jax
import jax.numpy as jnp
from jax import lax
from jax.experimental import pallas as pl
from jax.experimental.pallas import tpu as pltpu
from jax.experimental.pallas import tpu_sc as plsc

HIDDEN = 64
B, T = 4096, 200
NC, NS = 2, 16
NW = NC * NS
BT = B // NW
HG, HR, LN = 8, 8, 128

_mesh = plsc.VectorSubcoreMesh(core_axis_name="c", subcore_axis_name="s")


@functools.partial(
    pl.kernel,
    mesh=_mesh,
    compiler_params=pltpu.CompilerParams(
        use_tc_tiling_on_sc=False,
        needs_layout_passes=False,
        disable_bounds_checks=True,
    ),
    out_type=jax.ShapeDtypeStruct((T, HG, NW, HR, LN), jnp.float32),
    scratch_types=[
        pltpu.VMEM((BT * T,), jnp.int32),
        pltpu.VMEM((2, BT), jnp.int32),
        pltpu.VMEM((2, BT, HIDDEN), jnp.float32),
        pltpu.VMEM((2, HIDDEN, LN), jnp.float32),
        pltpu.SemaphoreType.DMA((2,)),
        pltpu.SemaphoreType.DMA((2,)),
    ],
)
def _embed(x_hbm, table_hbm, out5, xs, idxu, rows, tbuf, sem_g, sem_o):
    w = lax.axis_index("s") * NC + lax.axis_index("c")
    pltpu.sync_copy(x_hbm.at[pl.ds(w * BT * T, BT * T)], xs)

    iota = lax.iota(jnp.int32, 16)
    ib = [iota + blk * 16 for blk in range(8)]
    pa = [(iota + blk * 16) * T for blk in range(8)]

    def stage_idx(t, q):
        for blk in range(8):
            v = plsc.load_gather(xs, [pa[blk] + t])
            idxu[q, pl.ds(blk * 16, 16)] = v

    def gather_desc(q):
        return pltpu.make_async_copy(
            table_hbm.at[idxu.at[q]], rows.at[q], sem_g.at[q]
        )

    def out_descs(t, q):
        return [
            pltpu.make_async_copy(
                tbuf.at[q, pl.ds(hg * HR, HR), :],
                out5.at[t, hg, w],
                sem_o.at[q],
            )
            for hg in range(HG)
        ]

    def out_start(t, q):
        for d in out_descs(t, q):
            d.start()

    def out_wait(t, q):
        for d in out_descs(t, q):
            d.wait()

    def transpose(q):
        @plsc.parallel_loop(0, HIDDEN, 1, unroll=16)
        def _h_body(h):
            colv = jnp.full((16,), h, jnp.int32)
            for blk in range(8):
                v = plsc.load_gather(rows.at[q], [ib[blk], colv])
                tbuf[q, h, pl.ds(blk * 16, 16)] = v

    stage_idx(0, 0)
    gather_desc(0).start()
    stage_idx(1, 1)
    gather_desc(1).start()
    for q in range(2):
        gather_desc(q).wait()
        transpose(q)
        stage_idx(q + 2, q)
        gather_desc(q).start()
        out_start(q, q)

    def group(g, _):
        for q in range(2):
            t = g * 2 + q
            gather_desc(q).wait()
            out_wait(t - 2, q)
            transpose(q)
            stage_idx(t + 2, q)
            gather_desc(q).start()
            out_start(t, q)
        return 0

    lax.fori_loop(1, T // 2 - 1, group, 0)

    for q, t in ((0, T - 2), (1, T - 1)):
        gather_desc(q).wait()
        out_wait(t - 2, q)
        transpose(q)
        out_start(t, q)
    out_wait(T - 2, 0)
    out_wait(T - 1, 1)


def kernel(x, table):
    xf = x.reshape(-1).astype(jnp.int32)
    out5 = _embed(xf, table)
    y = out5.transpose(2, 4, 0, 1, 3)
    return y.reshape(B, T, HIDDEN)


# speedup vs baseline: 3.0593x; 3.0593x over previous
import functools

import jax
import jax.numpy as jnp
from jax import lax
from jax.experimental import pallas as pl
from jax.experimental.pallas import tpu as pltpu
from jax.experimental.pallas import tpu_sc as plsc

HIDDEN = 64
B, T = 4096, 200
NC, NS = 2, 16
NW = NC * NS
BT = B // NW
HG, HR, LN = 8, 8, 128

_mesh = plsc.VectorSubcoreMesh(core_axis_name="c", subcore_axis_name="s")


@functools.partial(
    pl.kernel,
    mesh=_mesh,
    compiler_params=pltpu.CompilerParams(
        use_tc_tiling_on_sc=False,
        needs_layout_passes=False,
        disable_bounds_checks=True,
    ),
    out_type=jax.ShapeDtypeStruct((T, HG, NW, HR, LN), jnp.float32),
    scratch_types=[
        pltpu.VMEM((BT * T,), jnp.int32),
        pltpu.VMEM((2, BT), jnp.int32),
        pltpu.VMEM((2, BT, HIDDEN), jnp.float32),
        pltpu.VMEM((2, HIDDEN, LN + 1), jnp.float32),
        pltpu.SemaphoreType.DMA((2,)),
        pltpu.SemaphoreType.DMA((2,)),
    ],
)
def _embed(x_hbm, table_hbm, out5, xs, idxu, rows, tbuf, sem_g, sem_o):
    w = lax.axis_index("s") * NC + lax.axis_index("c")
    pltpu.sync_copy(x_hbm.at[pl.ds(w * BT * T, BT * T)], xs)

    iota = lax.iota(jnp.int32, 16)
    ib = [iota + blk * 16 for blk in range(8)]
    pa = [(iota + blk * 16) * T for blk in range(8)]

    def stage_idx(t, q):
        for blk in range(8):
            v = plsc.load_gather(xs, [pa[blk] + t])
            idxu[q, pl.ds(blk * 16, 16)] = v

    def gather_desc(q):
        return pltpu.make_async_copy(
            table_hbm.at[idxu.at[q]], rows.at[q], sem_g.at[q]
        )

    def out_descs(t, q):
        return [
            pltpu.make_async_copy(
                tbuf.at[q, pl.ds(hg * HR, HR), pl.ds(0, LN)],
                out5.at[t, hg, w],
                sem_o.at[q],
            )
            for hg in range(HG)
        ]

    def out_start(t, q):
        for d in out_descs(t, q):
            d.start()

    def out_wait(t, q):
        for d in out_descs(t, q):
            d.wait()

    sidx = [iota + qq * 16 for qq in range(4)]

    def transpose(q):
        @plsc.parallel_loop(0, BT, 1, unroll=8)
        def _b_body(b):
            lanev = jnp.full((16,), b, jnp.int32)
            for qq in range(4):
                v = rows[q, b, pl.ds(qq * 16, 16)]
                plsc.store_scatter(tbuf.at[q], [sidx[qq], lanev], v)

    stage_idx(0, 0)
    gather_desc(0).start()
    stage_idx(1, 1)
    gather_desc(1).start()
    for q in range(2):
        gather_desc(q).wait()
        transpose(q)
        stage_idx(q + 2, q)
        gather_desc(q).start()
        out_start(q, q)

    def group(g, _):
        for q in range(2):
            t = g * 2 + q
            gather_desc(q).wait()
            out_wait(t - 2, q)
            transpose(q)
            stage_idx(t + 2, q)
            gather_desc(q).start()
            out_start(t, q)
        return 0

    lax.fori_loop(1, T // 2 - 1, group, 0)

    for q, t in ((0, T - 2), (1, T - 1)):
        gather_desc(q).wait()
        out_wait(t - 2, q)
        transpose(q)
        out_start(t, q)
    out_wait(T - 2, 0)
    out_wait(T - 1, 1)


def kernel(x, table):
    xf = x.reshape(-1).astype(jnp.int32)
    out5 = _embed(xf, table)
    y = out5.transpose(2, 4, 0, 1, 3)
    return y.reshape(B, T, HIDDEN)


# speedup vs baseline: 3.1474x; 1.0288x over previous
import functools

import jax
import jax.numpy as jnp
from jax import lax
from jax.experimental import pallas as pl
from jax.experimental.pallas import tpu as pltpu
from jax.experimental.pallas import tpu_sc as plsc

HIDDEN = 64
B, T = 4096, 200
NC, NS = 2, 16
NW = NC * NS
BT = B // NW
HG, HR, LN = 8, 8, 128
LNP = LN + 1

_mesh = plsc.VectorSubcoreMesh(core_axis_name="c", subcore_axis_name="s")


@functools.partial(
    pl.kernel,
    mesh=_mesh,
    compiler_params=pltpu.CompilerParams(
        use_tc_tiling_on_sc=False,
        needs_layout_passes=False,
        disable_bounds_checks=True,
    ),
    out_type=jax.ShapeDtypeStruct((T, HG, NW, HR, LN), jnp.float32),
    scratch_types=[
        pltpu.VMEM((T, BT), jnp.int32),
        pltpu.VMEM((2, BT, HIDDEN), jnp.float32),
        pltpu.VMEM((2, HG, HR, LNP), jnp.float32),
        pltpu.SemaphoreType.DMA((2,)),
        pltpu.SemaphoreType.DMA((2,)),
    ],
)
def _embed(xt_hbm, table_hbm, out5, xs, rows, tbuf, sem_g, sem_o):
    w = lax.axis_index("s") * NC + lax.axis_index("c")
    pltpu.sync_copy(xt_hbm.at[:, pl.ds(w * BT, BT)], xs)

    iota = lax.iota(jnp.int32, 16)
    hgv = [(iota + qq * 16) // HR for qq in range(4)]
    hrv = [(iota + qq * 16) % HR for qq in range(4)]

    def gather_desc(t, q):
        return pltpu.make_async_copy(
            table_hbm.at[xs.at[t]], rows.at[q], sem_g.at[q]
        )

    def out_desc(t, q):
        return pltpu.make_async_copy(
            tbuf.at[q, :, :, pl.ds(0, LN)], out5.at[t, :, w], sem_o.at[q]
        )

    def transpose(q):
        @plsc.parallel_loop(0, BT, 1, unroll=8)
        def _b_body(b):
            lanev = jnp.full((16,), b, jnp.int32)
            for qq in range(4):
                v = rows[q, b, pl.ds(qq * 16, 16)]
                plsc.store_scatter(tbuf.at[q], [hgv[qq], hrv[qq], lanev], v)

    gather_desc(0, 0).start()
    gather_desc(1, 1).start()
    for q in range(2):
        gather_desc(q, q).wait()
        transpose(q)
        gather_desc(q + 2, q).start()
        out_desc(q, q).start()

    def group(g, _):
        for q in range(2):
            t = g * 2 + q
            gather_desc(t, q).wait()
            out_desc(t - 2, q).wait()
            transpose(q)
            gather_desc(t + 2, q).start()
            out_desc(t, q).start()
        return 0

    lax.fori_loop(1, T // 2 - 1, group, 0)

    for q, t in ((0, T - 2), (1, T - 1)):
        gather_desc(t, q).wait()
        out_desc(t - 2, q).wait()
        transpose(q)
        out_desc(t, q).start()
    out_desc(T - 2, 0).wait()
    out_desc(T - 1, 1).wait()


def kernel(x, table):
    xt = x.T.astype(jnp.int32)
    out5 = _embed(xt, table)
    y = out5.transpose(2, 4, 0, 1, 3)
    return y.reshape(B, T, HIDDEN)


# speedup vs baseline: 3.4763x; 1.1045x over previous
import functools

import jax
import jax.numpy as jnp
from jax import lax
from jax.experimental import pallas as pl
from jax.experimental.pallas import tpu as pltpu
from jax.experimental.pallas import tpu_sc as plsc

HIDDEN = 64
B, T = 4096, 200
NC, NS = 2, 16
NW = NC * NS
BT = B // NW
HG, HR, LN = 8, 8, 128
LNP = LN + 1

_mesh = plsc.VectorSubcoreMesh(core_axis_name="c", subcore_axis_name="s")


@functools.partial(
    pl.kernel,
    mesh=_mesh,
    compiler_params=pltpu.CompilerParams(
        use_tc_tiling_on_sc=False,
        needs_layout_passes=False,
        disable_bounds_checks=True,
    ),
    out_type=jax.ShapeDtypeStruct((T, HG, NW, HR, LN), jnp.float32),
    scratch_types=[
        pltpu.VMEM((T, BT), jnp.int32),
        pltpu.VMEM((4, BT, HIDDEN), jnp.float32),
        pltpu.VMEM((2, HG, HR, LNP), jnp.float32),
        pltpu.SemaphoreType.DMA((4,)),
        pltpu.SemaphoreType.DMA((2,)),
    ],
)
def _embed(xt_hbm, table_hbm, out5, xs, rows, tbuf, sem_g, sem_o):
    w = lax.axis_index("s") * NC + lax.axis_index("c")
    pltpu.sync_copy(xt_hbm.at[:, pl.ds(w * BT, BT)], xs)

    iota = lax.iota(jnp.int32, 16)
    hgv = [(iota + qq * 16) // HR for qq in range(4)]
    hrv = [(iota + qq * 16) % HR for qq in range(4)]

    def gather_desc(t, q):
        return pltpu.make_async_copy(
            table_hbm.at[xs.at[t]], rows.at[q], sem_g.at[q]
        )

    def out_desc(t, q):
        return pltpu.make_async_copy(
            tbuf.at[q, :, :, pl.ds(0, LN)], out5.at[t, :, w], sem_o.at[q]
        )

    def transpose(r, q):
        @plsc.parallel_loop(0, BT, 1, unroll=8)
        def _b_body(b):
            lanev = jnp.full((16,), b, jnp.int32)
            for qq in range(4):
                v = rows[r, b, pl.ds(qq * 16, 16)]
                plsc.store_scatter(tbuf.at[q], [hgv[qq], hrv[qq], lanev], v)

    for r in range(4):
        gather_desc(r, r).start()
    for t0 in range(4):
        gather_desc(t0, t0).wait()
        transpose(t0 % 4, t0 % 2)
        if t0 >= 2:
            out_desc(t0 - 2, t0 % 2).wait()
        gather_desc(t0 + 4, t0 % 4).start()
        out_desc(t0, t0 % 2).start()

    def group(g, _):
        for r in range(4):
            t = g * 4 + r
            q = r % 2
            gather_desc(t, r).wait()
            out_desc(t - 2, q).wait()
            transpose(r, q)
            gather_desc(t + 4, r).start()
            out_desc(t, q).start()
        return 0

    lax.fori_loop(1, T // 4 - 1, group, 0)

    for t in range(T - 4, T):
        r, q = t % 4, t % 2
        gather_desc(t, r).wait()
        out_desc(t - 2, q).wait()
        transpose(r, q)
        out_desc(t, q).start()
    out_desc(T - 2, 0).wait()
    out_desc(T - 1, 1).wait()


def kernel(x, table):
    xt = x.T.astype(jnp.int32)
    out5 = _embed(xt, table)
    y = out5.transpose(2, 4, 0, 1, 3)
    return y.reshape(B, T, HIDDEN)
